# baseline (speedup 1.0000x reference)
import functools

import numpy as np
import jax
import jax.numpy as jnp
from jax import lax
from jax.experimental import pallas as pl
from jax.experimental.pallas import tpu as pltpu
from jax.experimental.pallas import tpu_sc as plsc

_MAXLEN = 2048
_D = 128
_B = 4
_BT = _B * _MAXLEN
_NC, _NS, _L = 2, 16, 16
_NW = _NC * _NS
_BPW = _BT // _NW
_CH = 64
_NCH = _BPW // _CH


def _positional_encoding():
    pos = np.arange(_MAXLEN)[:, np.newaxis]
    i = np.arange(_D)[np.newaxis, :]
    angle = pos * (1.0 / np.power(10000, 2 * (i // 2) / np.float32(_D)))
    angle[:, 0::2] = np.sin(angle[:, 0::2])
    angle[:, 1::2] = np.cos(angle[:, 1::2])
    return angle.astype(np.float32)


_POS = _positional_encoding()

_mesh = plsc.VectorSubcoreMesh(core_axis_name="c", subcore_axis_name="s")


@functools.partial(
    pl.kernel,
    mesh=_mesh,
    out_type=jax.ShapeDtypeStruct((_BT, _D), jnp.float32),
    scratch_types=[
        pltpu.VMEM((_NCH, _CH), jnp.int32),
        pltpu.VMEM((_BPW, _D), jnp.float32),
        [pltpu.SemaphoreType.DMA] * _NCH,
        [pltpu.SemaphoreType.DMA] * _NCH,
        pltpu.SemaphoreType.DMA,
    ],
)
def _emb_kernel(x_hbm, table_hbm, pos_hbm, out_hbm, idx_v, rows_v, psems, gsems, osem):
    wid = lax.axis_index("s") * _NC + lax.axis_index("c")
    base = wid * _BPW
    l0 = lax.rem(base, _MAXLEN)
    pos_h = []
    for c in range(_NCH):
        pos_h.append(
            pltpu.async_copy(
                pos_hbm.at[pl.ds(l0 + c * _CH, _CH)],
                rows_v.at[pl.ds(c * _CH, _CH)],
                psems[c],
            )
        )
    pltpu.sync_copy(x_hbm.at[pl.ds(wid * _NCH, _NCH)], idx_v)
    g_h = []
    for c in range(_NCH):
        pos_h[c].wait()
        g_h.append(
            pltpu.async_copy(
                table_hbm.at[idx_v.at[c]],
                rows_v.at[pl.ds(c * _CH, _CH)],
                gsems[c],
                add=True,
            )
        )
    o_h = []
    for c in range(_NCH):
        g_h[c].wait()
        o_h.append(
            pltpu.async_copy(
                rows_v.at[pl.ds(c * _CH, _CH)],
                out_hbm.at[pl.ds(base + c * _CH, _CH)],
                osem,
            )
        )
    for h in o_h:
        h.wait()


def kernel(x, table):
    idx = x.reshape(_BT // _CH, _CH).astype(jnp.int32)
    out = _emb_kernel(idx, table, jnp.asarray(_POS))
    return out.reshape(_B, _MAXLEN, _D)


# speedup vs baseline: 1.0277x; 1.0277x over previous
import functools

import numpy as np
import jax
import jax.numpy as jnp
from jax import lax
from jax.experimental import pallas as pl
from jax.experimental.pallas import tpu as pltpu
from jax.experimental.pallas import tpu_sc as plsc

_MAXLEN = 2048
_D = 128
_B = 4
_BT = _B * _MAXLEN
_NC, _NS, _L = 2, 16, 16
_NW = _NC * _NS
_LPW = _MAXLEN // _NW


def _positional_encoding():
    pos = np.arange(_MAXLEN)[:, np.newaxis]
    i = np.arange(_D)[np.newaxis, :]
    angle = pos * (1.0 / np.power(10000, 2 * (i // 2) / np.float32(_D)))
    angle[:, 0::2] = np.sin(angle[:, 0::2])
    angle[:, 1::2] = np.cos(angle[:, 1::2])
    return angle.astype(np.float32)


_POS = _positional_encoding()

_mesh = plsc.VectorSubcoreMesh(core_axis_name="c", subcore_axis_name="s")


@functools.partial(
    pl.kernel,
    mesh=_mesh,
    out_type=jax.ShapeDtypeStruct((_BT, _D), jnp.float32),
    scratch_types=[
        pltpu.VMEM((_B, _LPW), jnp.int32),
        pltpu.VMEM((_LPW, _D), jnp.float32),
        pltpu.VMEM((_B * _LPW, _D), jnp.float32),
        pltpu.SemaphoreType.DMA,
        [pltpu.SemaphoreType.DMA] * _B,
        pltpu.SemaphoreType.DMA,
    ],
)
def _emb_kernel(x_hbm, table_hbm, pos_hbm, out_hbm, idx_v, pos_v, rows_v,
                psem, gsems, osem):
    wid = lax.axis_index("s") * _NC + lax.axis_index("c")
    l0 = wid * _LPW
    ph = pltpu.async_copy(pos_hbm.at[pl.ds(l0, _LPW)], pos_v, psem)
    pltpu.sync_copy(x_hbm.at[:, wid], idx_v)
    g_h = []
    for b in range(_B):
        g_h.append(
            pltpu.async_copy(
                table_hbm.at[idx_v.at[b]],
                rows_v.at[pl.ds(b * _LPW, _LPW)],
                gsems[b],
            )
        )
    ph.wait()

    o_h = []
    for b in range(_B):
        g_h[b].wait()

        def add_row(i, carry, b=b):
            r = b * _LPW + i
            for j in range(_D // _L):
                s = pl.ds(j * _L, _L)
                rows_v[r, s] = rows_v[r, s] + pos_v[i, s]
            return carry

        lax.fori_loop(0, _LPW, add_row, 0)
        o_h.append(
            pltpu.async_copy(
                rows_v.at[pl.ds(b * _LPW, _LPW)],
                out_hbm.at[pl.ds(b * _MAXLEN + l0, _LPW)],
                osem,
            )
        )
    for h in o_h:
        h.wait()


def kernel(x, table):
    idx = x.reshape(_B, _NW, _LPW).astype(jnp.int32)
    out = _emb_kernel(idx, table, jnp.asarray(_POS))
    return out.reshape(_B, _MAXLEN, _D)
